# baseline (speedup 1.0000x reference)
import functools

import jax
import jax.numpy as jnp
from jax.experimental import pallas as pl

_ROW_BLOCK = 200


def _pre_kernel(x_ref, wsp_ref, wft_ref, psp_ref, pft_ref):
    x = x_ref[...]
    psp_ref[...] = jnp.dot(x, wsp_ref[...], preferred_element_type=jnp.float32)
    pft_ref[...] = jnp.dot(x, wft_ref[...], preferred_element_type=jnp.float32)


def _pass1_kernel(adj_ref, p_ref, b1_ref, w2_ref, v_ref):
    a = adj_ref[...].astype(jnp.bfloat16)
    h = jnp.dot(a, p_ref[...].astype(jnp.bfloat16),
                preferred_element_type=jnp.float32)
    h = jnp.maximum(h + b1_ref[...], 0.0)
    v_ref[...] = jnp.dot(h.astype(jnp.bfloat16),
                         w2_ref[...].astype(jnp.bfloat16),
                         preferred_element_type=jnp.float32)


def _pass2_kernel(adj_ref, v_ref, b2_ref, e_ref):
    a = adj_ref[...].astype(jnp.bfloat16)
    e_ref[...] = (
        jnp.dot(a, v_ref[...].astype(jnp.bfloat16),
                preferred_element_type=jnp.float32)
        + b2_ref[...]
    )


def _gate_kernel(esp_ref, eft_ref, wg_ref, bg_ref, wp_ref, bp_ref, rs_ref,
                 fused_ref, spsp_ref, spsh_ref, ftsh_ref, ftsp_ref, attn_ref):
    o = wg_ref.shape[0]
    esp = esp_ref[...]
    eft = eft_ref[...]
    sp_spec = esp[:, :o]
    sp_sh = esp[:, o:]
    ft_spec = eft[:, :o]
    ft_sh = eft[:, o:]
    spsp_ref[...] = sp_spec
    spsh_ref[...] = sp_sh
    ftsh_ref[...] = ft_sh
    ftsp_ref[...] = ft_spec

    wg = wg_ref[...]
    bg = bg_ref[0, 0]
    s0 = jnp.dot(sp_spec, wg, preferred_element_type=jnp.float32)
    s1 = jnp.dot(sp_sh, wg, preferred_element_type=jnp.float32)
    s2 = jnp.dot(ft_sh, wg, preferred_element_type=jnp.float32)
    s3 = jnp.dot(ft_spec, wg, preferred_element_type=jnp.float32)
    scores = jnp.concatenate([s0, s1, s2, s3], axis=1) + bg
    m = jnp.max(scores, axis=1, keepdims=True)
    e = jnp.exp(scores - m)
    attn = e / jnp.sum(e, axis=1, keepdims=True)
    attn_ref[...] = attn

    fused = (attn[:, 0:1] * sp_spec + attn[:, 1:2] * sp_sh
             + attn[:, 2:3] * ft_sh + attn[:, 3:4] * ft_spec)
    proj = jnp.dot(fused, wp_ref[...], preferred_element_type=jnp.float32)
    fused_ref[...] = rs_ref[0, 0] * (proj + bp_ref[...])


def kernel(features, spatial_graph, feature_graph, Ws1, bs1, Ws2, bs2,
           Wf1, bf1, Wf2, bf2, Wsh1, bsh1, Wsh2, bsh2, wg, bg, Wp, bp,
           res_scale):
    n, d = features.shape
    h = Ws1.shape[1]
    o = Ws2.shape[1]
    blk = _ROW_BLOCK
    nb = n // blk
    assert nb * blk == n

    f32 = jnp.float32
    Wsp1 = jnp.concatenate([Ws1, Wsh1], axis=1)
    Wft1 = jnp.concatenate([Wf1, Wsh1], axis=1)
    b_sp1 = jnp.concatenate([bs1, bsh1])[None, :]
    b_ft1 = jnp.concatenate([bf1, bsh1])[None, :]
    z = jnp.zeros((h, o), f32)
    W2sp = jnp.block([[Ws2, z], [z, Wsh2]])
    W2ft = jnp.block([[Wf2, z], [z, Wsh2]])
    b_sp2 = jnp.concatenate([bs2, bsh2])[None, :]
    b_ft2 = jnp.concatenate([bf2, bsh2])[None, :]

    full = lambda *shape: pl.BlockSpec(shape, lambda i: (0,) * len(shape))
    rows = lambda *shape: pl.BlockSpec(shape, lambda i: (i,) + (0,) * (len(shape) - 1))

    psp, pft = pl.pallas_call(
        _pre_kernel,
        grid=(1,),
        in_specs=[full(n, d), full(d, 2 * h), full(d, 2 * h)],
        out_specs=[full(n, 2 * h), full(n, 2 * h)],
        out_shape=[jax.ShapeDtypeStruct((n, 2 * h), f32)] * 2,
    )(features, Wsp1, Wft1)

    def gcn_pass1(adj, p, b1, w2):
        return pl.pallas_call(
            _pass1_kernel,
            grid=(nb,),
            in_specs=[rows(blk, n), full(n, 2 * h), full(1, 2 * h),
                      full(2 * h, 2 * o)],
            out_specs=rows(blk, 2 * o),
            out_shape=jax.ShapeDtypeStruct((n, 2 * o), f32),
        )(adj, p, b1, w2)

    def gcn_pass2(adj, v, b2):
        return pl.pallas_call(
            _pass2_kernel,
            grid=(nb,),
            in_specs=[rows(blk, n), full(n, 2 * o), full(1, 2 * o)],
            out_specs=rows(blk, 2 * o),
            out_shape=jax.ShapeDtypeStruct((n, 2 * o), f32),
        )(adj, v, b2)

    vsp = gcn_pass1(spatial_graph, psp, b_sp1, W2sp)
    vft = gcn_pass1(feature_graph, pft, b_ft1, W2ft)
    esp = gcn_pass2(spatial_graph, vsp, b_sp2)
    eft = gcn_pass2(feature_graph, vft, b_ft2)

    gate_out = pl.pallas_call(
        _gate_kernel,
        grid=(nb,),
        in_specs=[rows(blk, 2 * o), rows(blk, 2 * o), full(o, 1),
                  full(1, 1), full(o, o), full(1, o), full(1, 1)],
        out_specs=[rows(blk, o)] * 5 + [rows(blk, 4)],
        out_shape=[jax.ShapeDtypeStruct((n, o), f32)] * 5
        + [jax.ShapeDtypeStruct((n, 4), f32)],
    )(esp, eft, wg, bg[None, :], Wp, bp[None, :],
      res_scale[None, :])
    fused_out, sp_specific, sp_shared, ft_shared, ft_specific, attn = gate_out
    return (fused_out, sp_specific, sp_shared, ft_shared, ft_specific,
            attn[:, :, None])


# speedup vs baseline: 1.1518x; 1.1518x over previous
import jax
import jax.numpy as jnp
from jax.experimental import pallas as pl

_BLK1 = 400
_BLK2 = 200

_bf16 = jnp.bfloat16
_f32 = jnp.float32


def _pre_kernel(x_ref, wsp_ref, wft_ref, psp_ref, pft_ref):
    x = x_ref[...]
    psp_ref[...] = jnp.dot(
        x, wsp_ref[...], preferred_element_type=_f32).astype(_bf16)
    pft_ref[...] = jnp.dot(
        x, wft_ref[...], preferred_element_type=_f32).astype(_bf16)


def _pass1_kernel(adj_ref, p_ref, b1_ref, w2_ref, v_ref):
    a = adj_ref[...].astype(_bf16)
    h = jnp.dot(a, p_ref[...], preferred_element_type=_f32)
    h = jnp.maximum(h + b1_ref[...], 0.0)
    v_ref[...] = jnp.dot(h.astype(_bf16), w2_ref[...],
                         preferred_element_type=_f32).astype(_bf16)


def _pass2_kernel(adjs_ref, adjf_ref, vsp_ref, vft_ref, bsp2_ref, bft2_ref,
                  wg_ref, bg_ref, wp_ref, bp_ref, rs_ref,
                  fused_ref, spsp_ref, spsh_ref, ftsh_ref, ftsp_ref, attn_ref):
    o = wg_ref.shape[0]
    esp = jnp.dot(adjs_ref[...].astype(_bf16), vsp_ref[...],
                  preferred_element_type=_f32) + bsp2_ref[...]
    eft = jnp.dot(adjf_ref[...].astype(_bf16), vft_ref[...],
                  preferred_element_type=_f32) + bft2_ref[...]
    sp_spec = esp[:, :o]
    sp_sh = esp[:, o:]
    ft_spec = eft[:, :o]
    ft_sh = eft[:, o:]
    spsp_ref[...] = sp_spec
    spsh_ref[...] = sp_sh
    ftsh_ref[...] = ft_sh
    ftsp_ref[...] = ft_spec

    wg = wg_ref[...]
    s0 = jnp.dot(sp_spec, wg, preferred_element_type=_f32)
    s1 = jnp.dot(sp_sh, wg, preferred_element_type=_f32)
    s2 = jnp.dot(ft_sh, wg, preferred_element_type=_f32)
    s3 = jnp.dot(ft_spec, wg, preferred_element_type=_f32)
    scores = jnp.concatenate([s0, s1, s2, s3], axis=1) + bg_ref[0, 0]
    m = jnp.max(scores, axis=1, keepdims=True)
    e = jnp.exp(scores - m)
    attn = e / jnp.sum(e, axis=1, keepdims=True)
    attn_ref[...] = attn

    fused = (attn[:, 0:1] * sp_spec + attn[:, 1:2] * sp_sh
             + attn[:, 2:3] * ft_sh + attn[:, 3:4] * ft_spec)
    proj = jnp.dot(fused, wp_ref[...], preferred_element_type=_f32)
    fused_ref[...] = rs_ref[0, 0] * (proj + bp_ref[...])


def kernel(features, spatial_graph, feature_graph, Ws1, bs1, Ws2, bs2,
           Wf1, bf1, Wf2, bf2, Wsh1, bsh1, Wsh2, bsh2, wg, bg, Wp, bp,
           res_scale):
    n, d = features.shape
    h = Ws1.shape[1]
    o = Ws2.shape[1]
    nb1 = n // _BLK1
    nb2 = n // _BLK2
    assert nb1 * _BLK1 == n and nb2 * _BLK2 == n

    Wsp1 = jnp.concatenate([Ws1, Wsh1], axis=1)
    Wft1 = jnp.concatenate([Wf1, Wsh1], axis=1)
    b_sp1 = jnp.concatenate([bs1, bsh1])[None, :]
    b_ft1 = jnp.concatenate([bf1, bsh1])[None, :]
    z = jnp.zeros((h, o), _f32)
    W2sp = jnp.block([[Ws2, z], [z, Wsh2]]).astype(_bf16)
    W2ft = jnp.block([[Wf2, z], [z, Wsh2]]).astype(_bf16)
    b_sp2 = jnp.concatenate([bs2, bsh2])[None, :]
    b_ft2 = jnp.concatenate([bf2, bsh2])[None, :]

    full = lambda *shape: pl.BlockSpec(shape, lambda i: (0,) * len(shape))
    rows = lambda *shape: pl.BlockSpec(
        shape, lambda i: (i,) + (0,) * (len(shape) - 1))

    psp, pft = pl.pallas_call(
        _pre_kernel,
        grid=(1,),
        in_specs=[full(n, d), full(d, 2 * h), full(d, 2 * h)],
        out_specs=[full(n, 2 * h), full(n, 2 * h)],
        out_shape=[jax.ShapeDtypeStruct((n, 2 * h), _bf16)] * 2,
    )(features, Wsp1, Wft1)

    def gcn_pass1(adj, p, b1, w2):
        return pl.pallas_call(
            _pass1_kernel,
            grid=(nb1,),
            in_specs=[rows(_BLK1, n), full(n, 2 * h), full(1, 2 * h),
                      full(2 * h, 2 * o)],
            out_specs=rows(_BLK1, 2 * o),
            out_shape=jax.ShapeDtypeStruct((n, 2 * o), _bf16),
        )(adj, p, b1, w2)

    vsp = gcn_pass1(spatial_graph, psp, b_sp1, W2sp)
    vft = gcn_pass1(feature_graph, pft, b_ft1, W2ft)

    gate_out = pl.pallas_call(
        _pass2_kernel,
        grid=(nb2,),
        in_specs=[rows(_BLK2, n), rows(_BLK2, n),
                  full(n, 2 * o), full(n, 2 * o),
                  full(1, 2 * o), full(1, 2 * o),
                  full(o, 1), full(1, 1), full(o, o), full(1, o), full(1, 1)],
        out_specs=[rows(_BLK2, o)] * 5 + [rows(_BLK2, 4)],
        out_shape=[jax.ShapeDtypeStruct((n, o), _f32)] * 5
        + [jax.ShapeDtypeStruct((n, 4), _f32)],
    )(spatial_graph, feature_graph, vsp, vft, b_sp2, b_ft2,
      wg, bg[None, :], Wp, bp[None, :], res_scale[None, :])
    fused_out, sp_specific, sp_shared, ft_shared, ft_specific, attn = gate_out
    return (fused_out, sp_specific, sp_shared, ft_shared, ft_specific,
            attn[:, :, None])
